# baseline (speedup 1.0000x reference)
import functools

import jax
import jax.numpy as jnp
from jax.experimental import pallas as pl
from jax.experimental.pallas import tpu as pltpu

_LT = 128


@functools.lru_cache(maxsize=None)
def _bcast(bs, odim, seq_len, rep):
    nblk = bs // rep
    rem = seq_len - _LT

    def body(tile_ref, out_ref, sa, sb, sems_a, sems_b):
        ta = tile_ref[:, :_LT]
        tb = tile_ref[:, _LT:seq_len]
        for r in range(rep):
            sa[r, :, :] = ta
            sb[r, :, :] = tb
        handles = []
        for j in range(nblk):
            handles.append(
                pltpu.async_copy(
                    sa,
                    out_ref.at[pl.ds(j * rep, rep), :, pl.ds(0, _LT)],
                    sems_a.at[j],
                )
            )
            handles.append(
                pltpu.async_copy(
                    sb,
                    out_ref.at[pl.ds(j * rep, rep), :, pl.ds(_LT, rem)],
                    sems_b.at[j],
                )
            )
        for h in handles:
            h.wait()

    lanes_pad = ((seq_len + _LT - 1) // _LT) * _LT
    return pl.pallas_call(
        body,
        grid=(1,),
        in_specs=[pl.BlockSpec((odim, lanes_pad), lambda i: (0, 0))],
        out_specs=pl.BlockSpec(memory_space=pltpu.MemorySpace.HBM),
        out_shape=jax.ShapeDtypeStruct((bs, odim, seq_len), jnp.float32),
        scratch_shapes=[
            pltpu.VMEM((rep, odim, _LT), jnp.float32),
            pltpu.VMEM((rep, odim, rem), jnp.float32),
            pltpu.SemaphoreType.DMA((nblk,)),
            pltpu.SemaphoreType.DMA((nblk,)),
        ],
    )


def kernel(x, emb_table):
    bs, _, seq_len = x.shape
    emb_dim = emb_table.shape[1]
    lanes_pad = ((seq_len + _LT - 1) // _LT) * _LT
    tile = emb_table[:seq_len].reshape(emb_dim, seq_len)
    tilep = jnp.pad(tile, ((0, 0), (0, lanes_pad - seq_len)))
    return _bcast(bs, emb_dim, seq_len, 32)(tilep)


# speedup vs baseline: 1.0447x; 1.0447x over previous
import functools

import jax
import jax.numpy as jnp
from jax.experimental import pallas as pl


@functools.lru_cache(maxsize=None)
def _bcast(bs, odim, lanes_pad, blk):
    def body(tile_ref, out_ref):
        out_ref[...] = jnp.broadcast_to(
            tile_ref[...][None], (blk, odim, lanes_pad)
        )

    return pl.pallas_call(
        body,
        grid=(bs // blk,),
        in_specs=[pl.BlockSpec((odim, lanes_pad), lambda i: (0, 0))],
        out_specs=pl.BlockSpec((blk, odim, lanes_pad), lambda i: (i, 0, 0)),
        out_shape=jax.ShapeDtypeStruct((bs, odim, lanes_pad), jnp.float32),
    )


def kernel(x, emb_table):
    bs, _, seq_len = x.shape
    emb_dim = emb_table.shape[1]
    lanes_pad = ((seq_len + 127) // 128) * 128
    tile = emb_table[:seq_len].reshape(emb_dim, seq_len)
    tilep = jnp.pad(tile, ((0, 0), (0, lanes_pad - seq_len)))
    out = _bcast(bs, emb_dim, lanes_pad, 64)(tilep)
    return out[:, :, :seq_len]


# speedup vs baseline: 1.0589x; 1.0137x over previous
import functools

import jax
import jax.numpy as jnp
from jax.experimental import pallas as pl


@functools.lru_cache(maxsize=None)
def _bcast(bs, odim, lanes_pad, blk):
    def body(tile_ref, out_ref):
        out_ref[...] = jnp.broadcast_to(
            tile_ref[...][None], (blk, odim, lanes_pad)
        )

    return pl.pallas_call(
        body,
        grid=(bs // blk,),
        in_specs=[pl.BlockSpec((odim, lanes_pad), lambda i: (0, 0))],
        out_specs=pl.BlockSpec((blk, odim, lanes_pad), lambda i: (i, 0, 0)),
        out_shape=jax.ShapeDtypeStruct((bs, odim, lanes_pad), jnp.float32),
    )


def kernel(x, emb_table):
    bs, _, seq_len = x.shape
    emb_dim = emb_table.shape[1]
    lanes_pad = ((seq_len + 127) // 128) * 128
    tile = emb_table[:seq_len].reshape(emb_dim, seq_len)
    tilep = jnp.pad(tile, ((0, 0), (0, lanes_pad - seq_len)))
    out = _bcast(bs, emb_dim, lanes_pad, 128)(tilep)
    return out[:, :, :seq_len]
